# baseline (speedup 1.0000x reference)
import numpy as np

import jax
import jax.numpy as jnp
from jax import lax
from jax.experimental import pallas as pl

_H = 32
_W = 32
_OC1 = 8
_OC2 = 16
_FEAT = 16


def _build_conv1_bands(conv1_w):
    w = jnp.zeros((3 * _W, _W * _OC1), jnp.float32)
    xs = np.arange(_W)
    oc = np.arange(_OC1)
    for j in range(3):
        for dxi in range(3):
            xi = xs + dxi - 1
            valid = (xi >= 0) & (xi < _W)
            xv, xiv = xs[valid], xi[valid]
            rows = np.broadcast_to(j * _W + xiv[:, None], (xv.size, _OC1))
            cols = xv[:, None] * _OC1 + oc[None, :]
            vals = jnp.broadcast_to(conv1_w[:, 0, j, dxi][None, :],
                                    (xv.size, _OC1))
            w = w.at[rows, cols].set(vals)
    return w


def _build_conv2_bands(conv2_w):
    w = jnp.zeros((3, _W * _OC1, 16 * _OC2), jnp.float32)
    x2 = np.arange(16)
    ic = np.arange(_OC1)
    oc = np.arange(_OC2)
    for dy in range(3):
        for dx in range(3):
            x2p = x2 + dx - 1
            valid = (x2p >= 0) & (x2p < 16)
            x2v, x2pv = x2[valid], x2p[valid]
            rows = (2 * x2pv[:, None] * _OC1 + ic[None, :])[:, :, None]
            cols = (x2v[:, None] * _OC2 + oc[None, :])[:, None, :]
            rows = np.broadcast_to(rows, (x2v.size, _OC1, _OC2))
            cols = np.broadcast_to(cols, (x2v.size, _OC1, _OC2))
            vals = jnp.broadcast_to(conv2_w[:, :, dy, dx].T[None], (x2v.size, _OC1, _OC2))
            w = w.at[dy, rows, cols].set(vals)
    return w


def _cnn_kernel(x_ref, w1_ref, w2_ref, wfc_ref, b1_ref, b2_ref, bfc_ref,
                out_ref):
    nb = x_ref.shape[0]
    r = nb * _H
    f32 = jnp.float32
    zb = jnp.bfloat16(0)

    x32 = x_ref[...].reshape(r, _W).astype(jnp.bfloat16)
    iy = lax.broadcasted_iota(jnp.int32, (r, 1), 0) % _H
    xm1 = jnp.where(iy != 0, jnp.roll(x32, 1, axis=0), zb)
    xp1 = jnp.where(iy != _H - 1, jnp.roll(x32, -1, axis=0), zb)
    xcat = jnp.concatenate([xm1, x32, xp1], axis=1)

    z = jnp.dot(xcat, w1_ref[...], preferred_element_type=f32)
    zm = jnp.maximum(z, jnp.roll(z, -1, axis=0))
    xm = jnp.maximum(zm, jnp.roll(zm, -_OC1, axis=1))
    p = jnp.maximum(xm + b1_ref[...], 0.0).astype(jnp.bfloat16)

    pd = jnp.where(iy >= 2, jnp.roll(p, 2, axis=0), zb)
    pu = jnp.where(iy < _H - 2, jnp.roll(p, -2, axis=0), zb)
    e = (jnp.dot(pd, w2_ref[0], preferred_element_type=f32)
         + jnp.dot(p, w2_ref[1], preferred_element_type=f32)
         + jnp.dot(pu, w2_ref[2], preferred_element_type=f32))
    rr = jnp.maximum(e + b2_ref[...], 0.0)
    rrm = jnp.where(iy % 2 == 0, rr, 0.0).astype(jnp.bfloat16)

    t = jnp.dot(rrm, wfc_ref[...], preferred_element_type=f32)
    node = t.reshape(nb, _H, _FEAT).sum(axis=1) + bfc_ref[...]
    node = jnp.maximum(node, 0.0)
    out_ref[0] = jnp.mean(node.reshape(nb // 64, 64, _FEAT), axis=1)


def _head_kernel(m_ref, xl_ref, g1w_ref, g1b_ref, g2w_ref, g2b_ref,
                 f1w_ref, f1b_ref, f2w_ref, f2b_ref, out_ref):
    hp = lax.Precision.HIGHEST
    m = m_ref[...]
    h1 = jnp.maximum(jnp.dot(m, g1w_ref[...], precision=hp,
                             preferred_element_type=jnp.float32) + g1b_ref[...], 0.0)
    h2 = jnp.maximum(jnp.dot(h1, g2w_ref[...], precision=hp,
                             preferred_element_type=jnp.float32) + g2b_ref[...], 0.0)
    comb = jnp.concatenate([h2, xl_ref[...]], axis=1)
    o1 = jnp.maximum(jnp.dot(comb, f1w_ref[...], precision=hp,
                             preferred_element_type=jnp.float32) + f1b_ref[...], 0.0)
    out_ref[...] = jnp.dot(o1, f2w_ref[...], precision=hp,
                           preferred_element_type=jnp.float32) + f2b_ref[...]


def kernel(x_eeg, x_latent, conv1_w, conv1_b, conv2_w, conv2_b, cnn_fc_w, cnn_fc_b,
           gcn1_w, gcn1_b, gcn2_w, gcn2_b, fc1_w, fc1_b, fc2_w, fc2_b):
    b, c, h, w = x_eeg.shape
    n = b * c
    nb = 128
    steps = n // nb

    w1 = _build_conv1_bands(conv1_w).astype(jnp.bfloat16)
    w2 = _build_conv2_bands(conv2_w).astype(jnp.bfloat16)
    wfc = (jnp.tile(cnn_fc_w, (16, 1)) / 256.0).astype(jnp.bfloat16)
    b1l = jnp.tile(conv1_b, _W)[None]
    b2l = jnp.tile(conv2_b, 16)[None]

    xq = x_eeg.reshape(n, h, w)

    m_blocks = pl.pallas_call(
        _cnn_kernel,
        grid=(steps,),
        in_specs=[
            pl.BlockSpec((nb, h, w), lambda i: (i, 0, 0)),
            pl.BlockSpec(w1.shape, lambda i: (0, 0)),
            pl.BlockSpec(w2.shape, lambda i: (0, 0, 0)),
            pl.BlockSpec(wfc.shape, lambda i: (0, 0)),
            pl.BlockSpec(b1l.shape, lambda i: (0, 0)),
            pl.BlockSpec(b2l.shape, lambda i: (0, 0)),
            pl.BlockSpec((1, _FEAT), lambda i: (0, 0)),
        ],
        out_specs=pl.BlockSpec((1, nb // 64, _FEAT), lambda i: (i, 0, 0)),
        out_shape=jax.ShapeDtypeStruct((steps, nb // 64, _FEAT), jnp.float32),
    )(xq, w1, w2, wfc, b1l, b2l, cnn_fc_b[None])
    m_all = m_blocks.reshape(b, _FEAT)

    f2wp = jnp.zeros((fc2_w.shape[0], 8), jnp.float32).at[:, :fc2_w.shape[1]].set(fc2_w)
    f2bp = jnp.zeros((1, 8), jnp.float32).at[0, :fc2_b.shape[0]].set(fc2_b)
    out = pl.pallas_call(
        _head_kernel,
        out_shape=jax.ShapeDtypeStruct((b, 8), jnp.float32),
    )(m_all, x_latent, gcn1_w, gcn1_b[None], gcn2_w, gcn2_b[None],
      fc1_w, fc1_b[None], f2wp, f2bp)
    return out[:, :fc2_w.shape[1]]
